# baseline (speedup 1.0000x reference)
import functools

import jax
import jax.numpy as jnp
from jax import lax
from jax.experimental import pallas as pl
from jax.experimental.pallas import tpu as pltpu
from jax.experimental.pallas import tpu_sc as plsc

_N = 16384
_D = 512
_NC = 2
_NS = 16
_NW = _NC * _NS
_ROWS_PER_W = _N // _NW
_CHUNK = 64
_NCH = _ROWS_PER_W // _CHUNK


def _sc_body(x_hbm, o_hbm, buf0, buf1, isem0, isem1, osem0, osem1):
    wid = lax.axis_index("s") * _NC + lax.axis_index("c")
    base = wid * _ROWS_PER_W
    bufs = (buf0, buf1)
    isems = (isem0, isem1)
    osems = (osem0, osem1)

    def in_cp(i):
        b = i % 2
        return pltpu.make_async_copy(
            x_hbm.at[pl.ds(base + i * _CHUNK, _CHUNK)], bufs[b], isems[b]
        )

    def out_cp(i):
        b = i % 2
        return pltpu.make_async_copy(
            bufs[b], o_hbm.at[pl.ds(base + i * _CHUNK, _CHUNK)], osems[b]
        )

    in_cp(0).start()
    for i in range(_NCH):
        if i + 1 < _NCH:
            if i >= 1:
                out_cp(i - 1).wait()
            in_cp(i + 1).start()
        in_cp(i).wait()
        out_cp(i).start()
    if _NCH >= 2:
        out_cp(_NCH - 2).wait()
    out_cp(_NCH - 1).wait()


def kernel(x, question_mask, table_mask, column_mask):
    n, d = x.shape
    run = pl.kernel(
        _sc_body,
        mesh=plsc.VectorSubcoreMesh(core_axis_name="c", subcore_axis_name="s"),
        out_type=jax.ShapeDtypeStruct((n, d), x.dtype),
        scratch_types=[
            pltpu.VMEM((_CHUNK, _D), jnp.float32),
            pltpu.VMEM((_CHUNK, _D), jnp.float32),
            pltpu.SemaphoreType.DMA,
            pltpu.SemaphoreType.DMA,
            pltpu.SemaphoreType.DMA,
            pltpu.SemaphoreType.DMA,
        ],
    )
    return run(x)


# speedup vs baseline: 2.1484x; 2.1484x over previous
import jax
import jax.numpy as jnp
from jax.experimental import pallas as pl


def _copy_body(x_ref, o_ref):
    o_ref[...] = x_ref[...]


def kernel(x, question_mask, table_mask, column_mask):
    n, d = x.shape
    blk = 6144
    return pl.pallas_call(
        _copy_body,
        grid=(pl.cdiv(n, blk),),
        in_specs=[pl.BlockSpec((blk, d), lambda i: (i, 0))],
        out_specs=pl.BlockSpec((blk, d), lambda i: (i, 0)),
        out_shape=jax.ShapeDtypeStruct((n, d), x.dtype),
    )(x)


# speedup vs baseline: 2.1581x; 1.0045x over previous
import jax
import jax.numpy as jnp
from jax.experimental import pallas as pl


def _copy_body(x_ref, o_ref):
    o_ref[...] = x_ref[...]


def kernel(x, question_mask, table_mask, column_mask):
    n, d = x.shape
    blk = 7424
    return pl.pallas_call(
        _copy_body,
        grid=(pl.cdiv(n, blk),),
        in_specs=[pl.BlockSpec((blk, d), lambda i: (i, 0))],
        out_specs=pl.BlockSpec((blk, d), lambda i: (i, 0)),
        out_shape=jax.ShapeDtypeStruct((n, d), x.dtype),
    )(x)
